# baseline (speedup 1.0000x reference)
import jax
import jax.numpy as jnp
from jax import lax
from jax.experimental import pallas as pl

_K = 5
_RADIUS = 0.07
_H = 0.03
_EPS = 1e-12

_N = 2048
_BR = 1024


def _sumsq3(v, axis_slice):
    s0 = axis_slice(v, 0)
    s1 = axis_slice(v, 1)
    s2 = axis_slice(v, 2)
    return (s0 * s0 + s1 * s1) + s2 * s2


def _repulsion_kernel(rows_ref, colsT_ref, out_ref):
    rows = rows_ref[0]
    colsT = colsT_ref[0]

    inner = lax.dot_general(
        rows,
        colsT,
        (((1,), (0,)), ((), ())),
        precision=lax.Precision.DEFAULT,
        preferred_element_type=jnp.float32,
    )
    inner = jnp.float32(-2.0) * inner
    xr = _sumsq3(rows, lambda v, c: v[:, c : c + 1])
    xc = _sumsq3(colsT, lambda v, c: v[c : c + 1, :])
    metric = (xr + inner) + xc

    d2e = jnp.zeros((_BR, _N), dtype=jnp.float32)
    for c in range(3):
        diff = rows[:, c : c + 1] - colsT[c : c + 1, :]
        d2e = d2e + diff * diff

    col_idx = jax.lax.broadcasted_iota(jnp.int32, (_BR, _N), 1).astype(jnp.float32)
    inf = jnp.float32(jnp.inf)

    acc = jnp.zeros((_BR, 1), dtype=jnp.float32)
    for k in range(_K + 1):
        m = jnp.min(metric, axis=1, keepdims=True)
        first = jnp.min(
            jnp.where(metric == m, col_idx, jnp.float32(_N)),
            axis=1,
            keepdims=True,
        )
        sel = col_idx == first
        if k > 0:
            v = jnp.sum(
                jnp.where(sel, d2e, jnp.float32(0.0)), axis=1, keepdims=True
            )
            d = jnp.sqrt(jnp.maximum(v, jnp.float32(_EPS)))
            t = d / jnp.float32(_H)
            w = jnp.exp(-(t * t))
            acc = acc + (jnp.float32(_RADIUS) - d) * w
        if k < _K:
            metric = jnp.where(sel, inf, metric)

    out_ref[0] = acc


@jax.jit
def kernel(pred):
    b, n, _ = pred.shape
    predT = jnp.swapaxes(pred, 1, 2)
    row_sums = pl.pallas_call(
        _repulsion_kernel,
        grid=(b, n // _BR),
        in_specs=[
            pl.BlockSpec((1, _BR, 3), lambda bi, ri: (bi, ri, 0)),
            pl.BlockSpec((1, 3, n), lambda bi, ri: (bi, 0, 0)),
        ],
        out_specs=pl.BlockSpec((1, _BR, 1), lambda bi, ri: (bi, ri, 0)),
        out_shape=jax.ShapeDtypeStruct((b, n, 1), jnp.float32),
    )(pred, predT)
    return jnp.sum(row_sums[:, :, 0], axis=1) / jnp.float32(n * _K)


# speedup vs baseline: 1.0493x; 1.0493x over previous
import jax
import jax.numpy as jnp
from jax import lax
from jax.experimental import pallas as pl

_K = 5
_RADIUS = 0.07
_H = 0.03
_EPS = 1e-12

_N = 2048
_BR = 512


def _sumsq3(v, axis_slice):
    s0 = axis_slice(v, 0)
    s1 = axis_slice(v, 1)
    s2 = axis_slice(v, 2)
    return (s0 * s0 + s1 * s1) + s2 * s2


def _repulsion_kernel(rows_ref, colsT_ref, out_ref):
    rows = rows_ref[0]
    colsT = colsT_ref[0]

    inner = lax.dot_general(
        rows,
        colsT,
        (((1,), (0,)), ((), ())),
        precision=lax.Precision.DEFAULT,
        preferred_element_type=jnp.float32,
    )
    inner = jnp.float32(-2.0) * inner
    xr = _sumsq3(rows, lambda v, c: v[:, c : c + 1])
    xc = _sumsq3(colsT, lambda v, c: v[c : c + 1, :])
    metric = (xr + inner) + xc

    d2e = jnp.zeros((_BR, _N), dtype=jnp.float32)
    for c in range(3):
        diff = rows[:, c : c + 1] - colsT[c : c + 1, :]
        d2e = d2e + diff * diff

    col_idx = jax.lax.broadcasted_iota(jnp.int32, (_BR, _N), 1).astype(jnp.float32)
    inf = jnp.float32(jnp.inf)

    acc = jnp.zeros((_BR, 1), dtype=jnp.float32)
    for k in range(_K + 1):
        m = jnp.min(metric, axis=1, keepdims=True)
        first = jnp.min(
            jnp.where(metric == m, col_idx, jnp.float32(_N)),
            axis=1,
            keepdims=True,
        )
        sel = col_idx == first
        if k > 0:
            v = jnp.sum(
                jnp.where(sel, d2e, jnp.float32(0.0)), axis=1, keepdims=True
            )
            d = jnp.sqrt(jnp.maximum(v, jnp.float32(_EPS)))
            t = d / jnp.float32(_H)
            w = jnp.exp(-(t * t))
            acc = acc + (jnp.float32(_RADIUS) - d) * w
        if k < _K:
            metric = jnp.where(sel, inf, metric)

    out_ref[0] = acc


@jax.jit
def kernel(pred):
    b, n, _ = pred.shape
    predT = jnp.swapaxes(pred, 1, 2)
    row_sums = pl.pallas_call(
        _repulsion_kernel,
        grid=(b, n // _BR),
        in_specs=[
            pl.BlockSpec((1, _BR, 3), lambda bi, ri: (bi, ri, 0)),
            pl.BlockSpec((1, 3, n), lambda bi, ri: (bi, 0, 0)),
        ],
        out_specs=pl.BlockSpec((1, _BR, 1), lambda bi, ri: (bi, ri, 0)),
        out_shape=jax.ShapeDtypeStruct((b, n, 1), jnp.float32),
    )(pred, predT)
    return jnp.sum(row_sums[:, :, 0], axis=1) / jnp.float32(n * _K)
